# baseline (speedup 1.0000x reference)
import functools

import jax
import jax.numpy as jnp
from jax import lax
from jax.experimental import pallas as pl
from jax.experimental.pallas import tpu as pltpu
from jax.experimental.pallas import tpu_sc as plsc

NUM_BINS = 12
EPS = 1e-4
LANES = 16
NC = 2
NS = 16
NW = NC * NS
LUT = 128
CHUNK = 16384
NBUF = 2


def _build_sc_call(n, chunk=CHUNK, nbuf=NBUF):
    per_w = n // NW
    chunk = min(chunk, per_w)
    nchunks = per_w // chunk
    if nchunks < nbuf or nchunks % nbuf:
        nbuf = 1
    assert per_w * NW == n and nchunks * chunk == per_w

    mesh = plsc.VectorSubcoreMesh(
        core_axis_name="c", subcore_axis_name="s",
        num_cores=NC, num_subcores=NS)

    @functools.partial(
        pl.kernel,
        mesh=mesh,
        out_type=jax.ShapeDtypeStruct((n,), jnp.float32),
        compiler_params=pltpu.CompilerParams(needs_layout_passes=False),
        scratch_types=[
            pltpu.VMEM((LANES,), jnp.float32),
            pltpu.VMEM((LANES,), jnp.float32),
            pltpu.VMEM((LANES,), jnp.float32),
            pltpu.VMEM((LANES + LUT,), jnp.float32),
            pltpu.VMEM((LANES + LUT,), jnp.float32),
            pltpu.VMEM((LUT * LANES,), jnp.float32),
            pltpu.VMEM((LUT * LANES,), jnp.float32),
            [pltpu.VMEM((chunk,), jnp.float32) for _ in range(nbuf)],
            [pltpu.VMEM((chunk,), jnp.int32) for _ in range(nbuf)],
            [pltpu.VMEM((chunk,), jnp.float32) for _ in range(nbuf)],
            [pltpu.SemaphoreType.DMA for _ in range(nbuf)],
            [pltpu.SemaphoreType.DMA for _ in range(nbuf)],
        ],
    )
    def sc_standardize(z_hbm, k_hbm, e_hbm, a_hbm, b_hbm, out_hbm,
                       e_v, a_v, b_v, lut_a, lut_b, rep_a, rep_b,
                       zbs, kbs, obs, in_sems, out_sems):
        wid = lax.axis_index("s") * NC + lax.axis_index("c")
        base = wid * per_w

        pltpu.sync_copy(e_hbm, e_v)
        pltpu.sync_copy(a_hbm, a_v)
        pltpu.sync_copy(b_hbm, b_v)

        for t in range(LUT // LANES):
            kf = (jnp.full((LANES,), t * LANES, jnp.int32)
                  + lax.iota(jnp.int32, LANES)).astype(jnp.float32)
            bin_ = jnp.zeros((LANES,), jnp.int32)
            for j in range(NUM_BINS - 1):
                ej = plsc.load_gather(
                    e_v, [jnp.full((LANES,), j + 1, jnp.int32)])
                bin_ = bin_ + (ej < kf).astype(jnp.int32)
            lut_a[pl.ds(LANES + t * LANES, LANES)] = plsc.load_gather(
                a_v, [bin_])
            lut_b[pl.ds(LANES + t * LANES, LANES)] = plsc.load_gather(
                b_v, [bin_])

        @plsc.parallel_loop(0, LUT, unroll=8)
        def _rep(k):
            src = jnp.zeros((LANES,), jnp.int32) + (LANES + k)
            rep_a[pl.ds(k * LANES, LANES)] = plsc.load_gather(lut_a, [src])
            rep_b[pl.ds(k * LANES, LANES)] = plsc.load_gather(lut_b, [src])

        def start_in(c, b):
            off = base + c * chunk
            pltpu.async_copy(z_hbm.at[pl.ds(off, chunk)], zbs[b], in_sems[b])
            pltpu.async_copy(k_hbm.at[pl.ds(off, chunk)], kbs[b], in_sems[b])

        def wait_in(c, b):
            off = base + c * chunk
            pltpu.make_async_copy(
                z_hbm.at[pl.ds(off, chunk)], zbs[b], in_sems[b]).wait()
            pltpu.make_async_copy(
                k_hbm.at[pl.ds(off, chunk)], kbs[b], in_sems[b]).wait()

        def start_out(c, b):
            off = base + c * chunk
            pltpu.async_copy(obs[b], out_hbm.at[pl.ds(off, chunk)],
                             out_sems[b])

        def wait_out(b):
            pltpu.make_async_copy(
                obs[b], out_hbm.at[pl.ds(base, chunk)], out_sems[b]).wait()

        def compute(b):
            zb, kb, ob = zbs[b], kbs[b], obs[b]

            @plsc.parallel_loop(0, chunk // LANES, unroll=32)
            def _vec(i):
                s = pl.ds(i * LANES, LANES)
                kv = kb[s]
                idx = kv * LANES + lax.iota(jnp.int32, LANES)
                a = plsc.load_gather(rep_a, [idx])
                b_ = plsc.load_gather(rep_b, [idx])
                ob[s] = zb[s] * a + b_

        if nbuf == 1:
            @pl.loop(0, nchunks)
            def _chunk(c):
                start_in(c, 0)
                wait_in(c, 0)
                compute(0)
                start_out(c, 0)
                wait_out(0)
            return

        for i in range(nbuf - 1):
            start_in(i, i)

        @pl.loop(0, nchunks, step=nbuf)
        def _ring(c):
            for db in range(nbuf):
                cc = c + db
                pf = cc + nbuf - 1
                pb = (db + nbuf - 1) % nbuf
                if db == 0:
                    start_in(pf, pb)
                else:
                    @pl.when(pf < nchunks)
                    def _():
                        start_in(pf, pb)
                wait_in(cc, db)

                @pl.when(cc >= nbuf)
                def _():
                    wait_out(db)

                compute(db)
                start_out(cc, db)

        for b in range(nbuf):
            wait_out(b)

    return sc_standardize


def kernel(Z_raw, K, bin_edges, running_mean, running_var):
    inner = bin_edges[1:NUM_BINS]
    sigma = jnp.sqrt(jnp.clip(running_var, EPS * EPS, None) + EPS)
    scale = 1.0 / sigma
    shift = -running_mean * scale
    edges16 = jnp.pad(inner, (1, LANES - NUM_BINS))
    scale16 = jnp.pad(scale, (0, LANES - NUM_BINS))
    shift16 = jnp.pad(shift, (0, LANES - NUM_BINS))
    sc_call = _build_sc_call(Z_raw.shape[0])
    return sc_call(Z_raw, K, edges16, scale16, shift16)


# speedup vs baseline: 1.0436x; 1.0436x over previous
import functools

import jax
import jax.numpy as jnp
from jax import lax
from jax.experimental import pallas as pl
from jax.experimental.pallas import tpu as pltpu
from jax.experimental.pallas import tpu_sc as plsc

NUM_BINS = 12
EPS = 1e-4
LANES = 16
NC = 2
NS = 16
NW = NC * NS
LUT = 128
CHUNK = 16384
NBUF = 2


def _build_sc_call(n, chunk=CHUNK, nbuf=NBUF):
    per_w = n // NW
    chunk = min(chunk, per_w)
    nchunks = per_w // chunk
    if nchunks < nbuf or nchunks % nbuf:
        nbuf = 1
    assert per_w * NW == n and nchunks * chunk == per_w

    mesh = plsc.VectorSubcoreMesh(
        core_axis_name="c", subcore_axis_name="s",
        num_cores=NC, num_subcores=NS)

    @functools.partial(
        pl.kernel,
        mesh=mesh,
        out_type=jax.ShapeDtypeStruct((n,), jnp.float32),
        compiler_params=pltpu.CompilerParams(needs_layout_passes=False),
        scratch_types=[
            pltpu.VMEM((LANES,), jnp.float32),
            pltpu.VMEM((LANES,), jnp.float32),
            pltpu.VMEM((LANES,), jnp.float32),
            pltpu.VMEM((LANES + LUT,), jnp.float32),
            pltpu.VMEM((LANES + LUT,), jnp.float32),
            pltpu.VMEM((LUT * LANES,), jnp.float32),
            pltpu.VMEM((LUT * LANES,), jnp.float32),
            [pltpu.VMEM((chunk,), jnp.float32) for _ in range(nbuf)],
            [pltpu.VMEM((chunk,), jnp.int32) for _ in range(nbuf)],
            [pltpu.VMEM((chunk,), jnp.float32) for _ in range(nbuf)],
            [pltpu.SemaphoreType.DMA for _ in range(nbuf)],
            [pltpu.SemaphoreType.DMA for _ in range(nbuf)],
        ],
    )
    def sc_standardize(z_hbm, k_hbm, e_hbm, a_hbm, b_hbm, out_hbm,
                       e_v, a_v, b_v, lut_a, lut_b, rep_a, rep_b,
                       zbs, kbs, obs, in_sems, out_sems):
        wid = lax.axis_index("s") * NC + lax.axis_index("c")
        base = wid * per_w

        pltpu.sync_copy(e_hbm, e_v)
        pltpu.sync_copy(a_hbm, a_v)
        pltpu.sync_copy(b_hbm, b_v)

        for t in range(LUT // LANES):
            kf = (jnp.full((LANES,), t * LANES, jnp.int32)
                  + lax.iota(jnp.int32, LANES)).astype(jnp.float32)
            bin_ = jnp.zeros((LANES,), jnp.int32)
            for j in range(NUM_BINS - 1):
                ej = plsc.load_gather(
                    e_v, [jnp.full((LANES,), j + 1, jnp.int32)])
                bin_ = bin_ + (ej < kf).astype(jnp.int32)
            lut_a[pl.ds(LANES + t * LANES, LANES)] = plsc.load_gather(
                a_v, [bin_])
            lut_b[pl.ds(LANES + t * LANES, LANES)] = plsc.load_gather(
                b_v, [bin_])

        @plsc.parallel_loop(0, LUT, unroll=8)
        def _rep(k):
            src = jnp.zeros((LANES,), jnp.int32) + (LANES + k)
            rep_a[pl.ds(k * LANES, LANES)] = plsc.load_gather(lut_a, [src])
            rep_b[pl.ds(k * LANES, LANES)] = plsc.load_gather(lut_b, [src])

        def start_in(c, b):
            off = base + c * chunk
            pltpu.async_copy(z_hbm.at[pl.ds(off, chunk)], zbs[b], in_sems[b])
            pltpu.async_copy(k_hbm.at[pl.ds(off, chunk)], kbs[b], in_sems[b])

        def wait_in(c, b):
            off = base + c * chunk
            pltpu.make_async_copy(
                z_hbm.at[pl.ds(off, chunk)], zbs[b], in_sems[b]).wait()
            pltpu.make_async_copy(
                k_hbm.at[pl.ds(off, chunk)], kbs[b], in_sems[b]).wait()

        def start_out(c, b):
            off = base + c * chunk
            pltpu.async_copy(obs[b], out_hbm.at[pl.ds(off, chunk)],
                             out_sems[b])

        def wait_out(b):
            pltpu.make_async_copy(
                obs[b], out_hbm.at[pl.ds(base, chunk)], out_sems[b]).wait()

        def compute(b):
            zb, kb, ob = zbs[b], kbs[b], obs[b]

            @plsc.parallel_loop(0, chunk // LANES, unroll=16)
            def _vec(i):
                s = pl.ds(i * LANES, LANES)
                kv = kb[s]
                idx = kv * LANES + lax.iota(jnp.int32, LANES)
                a = plsc.load_gather(rep_a, [idx])
                b_ = plsc.load_gather(rep_b, [idx])
                ob[s] = zb[s] * a + b_

        if nbuf == 1:
            @pl.loop(0, nchunks)
            def _chunk(c):
                start_in(c, 0)
                wait_in(c, 0)
                compute(0)
                start_out(c, 0)
                wait_out(0)
            return

        for i in range(nbuf - 1):
            start_in(i, i)

        @pl.loop(0, nchunks, step=nbuf)
        def _ring(c):
            for db in range(nbuf):
                cc = c + db
                pf = cc + nbuf - 1
                pb = (db + nbuf - 1) % nbuf
                if db == 0:
                    start_in(pf, pb)
                else:
                    @pl.when(pf < nchunks)
                    def _():
                        start_in(pf, pb)
                wait_in(cc, db)

                @pl.when(cc >= nbuf)
                def _():
                    wait_out(db)

                compute(db)
                start_out(cc, db)

        for b in range(nbuf):
            wait_out(b)

    return sc_standardize


def kernel(Z_raw, K, bin_edges, running_mean, running_var):
    inner = bin_edges[1:NUM_BINS]
    sigma = jnp.sqrt(jnp.clip(running_var, EPS * EPS, None) + EPS)
    scale = 1.0 / sigma
    shift = -running_mean * scale
    edges16 = jnp.pad(inner, (1, LANES - NUM_BINS))
    scale16 = jnp.pad(scale, (0, LANES - NUM_BINS))
    shift16 = jnp.pad(shift, (0, LANES - NUM_BINS))
    sc_call = _build_sc_call(Z_raw.shape[0])
    return sc_call(Z_raw, K, edges16, scale16, shift16)
